# baseline (speedup 1.0000x reference)
import jax
import jax.numpy as jnp
from jax.experimental import pallas as pl

_N_ROWS = 10000
_D = 128
_BLOCK_ROWS = 10000


def _ln_kernel(x_ref, w_ref, b_ref, o_ref):
    x = x_ref[...]
    mu = jnp.mean(x, axis=-1, keepdims=True)
    xc = x - mu
    var = jnp.mean(xc * xc, axis=-1, keepdims=True)
    o_ref[...] = xc * jax.lax.rsqrt(var + 1e-5) * w_ref[...] + b_ref[...]


def kernel(x, edge_index, x_time, edge_feature, edge_time, ln_weight, ln_bias):
    w = ln_weight.reshape(1, _D)
    b = ln_bias.reshape(1, _D)
    grid = _N_ROWS // _BLOCK_ROWS
    out = pl.pallas_call(
        _ln_kernel,
        grid=(grid,),
        in_specs=[
            pl.BlockSpec((_BLOCK_ROWS, _D), lambda i: (i, 0)),
            pl.BlockSpec((1, _D), lambda i: (0, 0)),
            pl.BlockSpec((1, _D), lambda i: (0, 0)),
        ],
        out_specs=pl.BlockSpec((_BLOCK_ROWS, _D), lambda i: (i, 0)),
        out_shape=jax.ShapeDtypeStruct((_N_ROWS, _D), x.dtype),
    )(x, w, b)
    return out


# speedup vs baseline: 1.0554x; 1.0554x over previous
import jax
import jax.numpy as jnp
from jax.experimental import pallas as pl

_N_ROWS = 10000
_D = 128
_BLOCK_ROWS = 3336


def _ln_kernel(x_ref, w_ref, b_ref, o_ref):
    x = x_ref[...]
    mu = jnp.mean(x, axis=-1, keepdims=True)
    xc = x - mu
    var = jnp.mean(xc * xc, axis=-1, keepdims=True)
    o_ref[...] = xc * jax.lax.rsqrt(var + 1e-5) * w_ref[...] + b_ref[...]


def kernel(x, edge_index, x_time, edge_feature, edge_time, ln_weight, ln_bias):
    w = ln_weight.reshape(1, _D)
    b = ln_bias.reshape(1, _D)
    grid = -(-_N_ROWS // _BLOCK_ROWS)
    out = pl.pallas_call(
        _ln_kernel,
        grid=(grid,),
        in_specs=[
            pl.BlockSpec((_BLOCK_ROWS, _D), lambda i: (i, 0)),
            pl.BlockSpec((1, _D), lambda i: (0, 0)),
            pl.BlockSpec((1, _D), lambda i: (0, 0)),
        ],
        out_specs=pl.BlockSpec((_BLOCK_ROWS, _D), lambda i: (i, 0)),
        out_shape=jax.ShapeDtypeStruct((_N_ROWS, _D), x.dtype),
    )(x, w, b)
    return out


# speedup vs baseline: 1.0830x; 1.0261x over previous
import jax
import jax.numpy as jnp
from jax.experimental import pallas as pl

_N_ROWS = 10000
_D = 128
_BLOCK_ROWS = 5000


def _ln_kernel(x_ref, w_ref, b_ref, o_ref):
    x = x_ref[...]
    mu = jnp.mean(x, axis=-1, keepdims=True)
    ex2 = jnp.mean(x * x, axis=-1, keepdims=True)
    var = ex2 - mu * mu
    scale = jax.lax.rsqrt(var + 1e-5)
    o_ref[...] = (x - mu) * scale * w_ref[...] + b_ref[...]


def kernel(x, edge_index, x_time, edge_feature, edge_time, ln_weight, ln_bias):
    w = ln_weight.reshape(1, _D)
    b = ln_bias.reshape(1, _D)
    grid = -(-_N_ROWS // _BLOCK_ROWS)
    out = pl.pallas_call(
        _ln_kernel,
        grid=(grid,),
        in_specs=[
            pl.BlockSpec((_BLOCK_ROWS, _D), lambda i: (i, 0)),
            pl.BlockSpec((1, _D), lambda i: (0, 0)),
            pl.BlockSpec((1, _D), lambda i: (0, 0)),
        ],
        out_specs=pl.BlockSpec((_BLOCK_ROWS, _D), lambda i: (i, 0)),
        out_shape=jax.ShapeDtypeStruct((_N_ROWS, _D), x.dtype),
    )(x, w, b)
    return out
